# Initial kernel scaffold; baseline (speedup 1.0000x reference)
import jax
import jax.numpy as jnp
from jax.experimental import pallas as pl


def kernel(user_emb_a, item_emb_a, user_emb_b, item_emb_b, adj_a_val, adj_b_val, adj_a_idx, adj_b_idx, data_a, data_b):
    raise NotImplementedError("write your pallas kernel here")


# speedup vs baseline: 1.0031x; 1.0031x over previous
import jax
import jax.numpy as jnp
from jax.experimental import pallas as pl

N_USER = 25000
N_OVERLAP = 10000
EMB = 64
LAYERS = 3
N = 50000
LAMBDA_A = 0.7
LAMBDA_B = 0.7


def _combine_body(side_ref, ego_ref, out_ref):
    s = side_ref[...]
    e = ego_ref[...]
    out_ref[...] = s + e * s


def _combine(side, ego):
    blk = 1000
    return pl.pallas_call(
        _combine_body,
        out_shape=jax.ShapeDtypeStruct((N, EMB), jnp.float32),
        grid=(N // blk,),
        in_specs=[
            pl.BlockSpec((blk, EMB), lambda i: (i, 0)),
            pl.BlockSpec((blk, EMB), lambda i: (i, 0)),
        ],
        out_specs=pl.BlockSpec((blk, EMB), lambda i: (i, 0)),
    )(side, ego)


def _spmm(idx, val, x):
    gathered = val[:, None] * jnp.take(x, idx[1], axis=0)
    return jax.ops.segment_sum(gathered, idx[0], num_segments=N)


def _transfer(egoA, egoB):
    ua, ia = egoA[:N_USER], egoA[N_USER:]
    ub, ib = egoB[:N_USER], egoB[N_USER:]
    oua, dua = ua[:N_OVERLAP], ua[N_OVERLAP:]
    oub, dub = ub[:N_OVERLAP], ub[N_OVERLAP:]
    u_lap = 0.5 * oua + 0.5 * oub
    ua_lam = LAMBDA_A * oua + (1.0 - LAMBDA_A) * oub
    ub_lam = LAMBDA_B * oub + (1.0 - LAMBDA_B) * oua
    new_oua = (u_lap + ua_lam) / 2.0
    new_oub = (u_lap + ub_lam) / 2.0
    egoA = jnp.concatenate([new_oua, dua, ia], axis=0)
    egoB = jnp.concatenate([new_oub, dub, ib], axis=0)
    return egoA, egoB


def kernel(user_emb_a, item_emb_a, user_emb_b, item_emb_b,
           adj_a_val, adj_b_val, adj_a_idx, adj_b_idx, data_a, data_b):
    egoA = jnp.concatenate([user_emb_a, item_emb_a], axis=0)
    egoB = jnp.concatenate([user_emb_b, item_emb_b], axis=0)
    sumA, sumB = egoA, egoB
    for _ in range(LAYERS):
        egoA = _combine(_spmm(adj_a_idx, adj_a_val, egoA), egoA)
        egoB = _combine(_spmm(adj_b_idx, adj_b_val, egoB), egoB)
        egoA, egoB = _transfer(egoA, egoB)
        sumA = sumA + egoA
        sumB = sumB + egoB
    embsA = sumA / (LAYERS + 1)
    embsB = sumB / (LAYERS + 1)
    ua, ia = embsA[:N_USER], embsA[N_USER:]
    ub, ib = embsB[:N_USER], embsB[N_USER:]
    pos_a = jnp.sum(ua[data_a[0]] * ia[data_a[1]], axis=-1)
    neg_a = jnp.sum(ua[data_a[0]] * ia[data_a[2]], axis=-1)
    loss = jnp.mean(jax.nn.softplus(neg_a - pos_a))
    pos_b = jnp.sum(ub[data_b[0]] * ib[data_b[1]], axis=-1)
    neg_b = jnp.sum(ub[data_b[0]] * ib[data_b[2]], axis=-1)
    loss = loss + jnp.mean(jax.nn.softplus(neg_b - pos_b))
    return loss


# speedup vs baseline: 2.6098x; 2.6018x over previous
import functools

import jax
import jax.numpy as jnp
from jax import lax
from jax.experimental import pallas as pl
from jax.experimental.pallas import tpu as pltpu
from jax.experimental.pallas import tpu_sc as plsc

N_USER = 25000
N_OVERLAP = 10000
EMB = 64
LAYERS = 3
N = 50000
LAMBDA_A = 0.7
LAMBDA_B = 0.7

NNZ = 800000
HALF = 32
CHUNK = 128
NS = 16
NC = 2
NCT = 391
NNZ_PAD = NS * NCT * CHUNK
RPT = 3000


def _make_spmm_sc():
    mesh = plsc.VectorSubcoreMesh(core_axis_name="c", subcore_axis_name="s",
                                  num_cores=NC, num_subcores=NS)

    @functools.partial(
        pl.kernel,
        out_type=jax.ShapeDtypeStruct((NC * N, HALF), jnp.float32),
        mesh=mesh,
        compiler_params=pltpu.CompilerParams(use_tc_tiling_on_sc=False),
        scratch_types=[
            pltpu.VMEM_SHARED((N, HALF), jnp.float32),
            pltpu.VMEM((1, CHUNK), jnp.int32),
            pltpu.VMEM((1, CHUNK), jnp.int32),
            pltpu.VMEM((1, CHUNK), jnp.float32),
            pltpu.VMEM((CHUNK, HALF), jnp.float32),
            pltpu.SemaphoreType.DMA,
        ],
    )
    def spmm(ego_hbm, cols_hbm, rows_hbm, vals_hbm, zeros_hbm, out_hbm,
             acc, cb, rb, vb, db, sem):
        kc = lax.axis_index("c")
        t = lax.axis_index("s")
        r0 = t * RPT
        pltpu.sync_copy(zeros_hbm.at[pl.ds(r0, RPT)], acc.at[pl.ds(r0, RPT)])

        @pl.when(t < 2)
        def _():
            rx = NS * RPT + t * 1000
            pltpu.sync_copy(zeros_hbm.at[pl.ds(rx, 1000)],
                            acc.at[pl.ds(rx, 1000)])
        plsc.subcore_barrier()
        koff = kc * N

        def chunk_body(j, _):
            off = (t * NCT + j) * CHUNK
            pltpu.sync_copy(cols_hbm.at[pl.ds(off, CHUNK)], cb.at[0])
            pltpu.sync_copy(rows_hbm.at[pl.ds(off, CHUNK)], rb.at[0])
            pltpu.sync_copy(vals_hbm.at[pl.ds(off, CHUNK)], vb.at[0])

            def addk(i, _):
                cb[0, pl.ds(i * 16, 16)] = cb[0, pl.ds(i * 16, 16)] + koff
                return 0
            lax.fori_loop(0, CHUNK // 16, addk, 0)
            pltpu.async_copy(ego_hbm.at[cb.at[0]], db, sem).wait()

            def grp(g, _):
                v = vb[0, pl.ds(g * 16, 16)]
                for j in range(16):
                    e = g * 16 + j
                    s = v[j]
                    db[e, pl.ds(0, 16)] = db[e, pl.ds(0, 16)] * s
                    db[e, pl.ds(16, 16)] = db[e, pl.ds(16, 16)] * s
                return 0
            lax.fori_loop(0, CHUNK // 16, grp, 0)
            pltpu.sync_copy(db, acc.at[rb.at[0]], add=True)
            return 0
        lax.fori_loop(0, NCT, chunk_body, 0)
        plsc.subcore_barrier()
        pltpu.sync_copy(acc.at[pl.ds(r0, RPT)],
                        out_hbm.at[pl.ds(koff + r0, RPT)])

        @pl.when(t < 2)
        def _():
            rx = NS * RPT + t * 1000
            pltpu.sync_copy(acc.at[pl.ds(rx, 1000)],
                            out_hbm.at[pl.ds(koff + rx, 1000)])

    return spmm


_spmm_sc = _make_spmm_sc()


def _combine_body(side_ref, ego_ref, out_ref):
    s = side_ref[...]
    e = ego_ref[...]
    out_ref[...] = s + e * s


def _combine(side, ego):
    blk = 1000
    return pl.pallas_call(
        _combine_body,
        out_shape=jax.ShapeDtypeStruct((N, EMB), jnp.float32),
        grid=(N // blk,),
        in_specs=[
            pl.BlockSpec((blk, EMB), lambda i: (i, 0)),
            pl.BlockSpec((blk, EMB), lambda i: (i, 0)),
        ],
        out_specs=pl.BlockSpec((blk, EMB), lambda i: (i, 0)),
    )(side, ego)


def _spmm(cols, rows, vals, ego, zeros):
    ego_flat = jnp.concatenate([ego[:, :HALF], ego[:, HALF:]], axis=0)
    out = _spmm_sc(ego_flat, cols, rows, vals, zeros)
    return jnp.concatenate([out[:N], out[N:]], axis=1)


def _transfer(egoA, egoB):
    ua, ia = egoA[:N_USER], egoA[N_USER:]
    ub, ib = egoB[:N_USER], egoB[N_USER:]
    oua, dua = ua[:N_OVERLAP], ua[N_OVERLAP:]
    oub, dub = ub[:N_OVERLAP], ub[N_OVERLAP:]
    u_lap = 0.5 * oua + 0.5 * oub
    ua_lam = LAMBDA_A * oua + (1.0 - LAMBDA_A) * oub
    ub_lam = LAMBDA_B * oub + (1.0 - LAMBDA_B) * oua
    new_oua = (u_lap + ua_lam) / 2.0
    new_oub = (u_lap + ub_lam) / 2.0
    egoA = jnp.concatenate([new_oua, dua, ia], axis=0)
    egoB = jnp.concatenate([new_oub, dub, ib], axis=0)
    return egoA, egoB


def _pad_edges(idx, val):
    pad = NNZ_PAD - NNZ
    cols = jnp.concatenate([idx[1].astype(jnp.int32),
                            jnp.zeros((pad,), jnp.int32)])
    rows = jnp.concatenate([idx[0].astype(jnp.int32),
                            (jnp.arange(pad, dtype=jnp.int32) * 64) % N])
    vals = jnp.concatenate([val, jnp.zeros((pad,), jnp.float32)])
    return cols, rows, vals


def kernel(user_emb_a, item_emb_a, user_emb_b, item_emb_b,
           adj_a_val, adj_b_val, adj_a_idx, adj_b_idx, data_a, data_b):
    egoA = jnp.concatenate([user_emb_a, item_emb_a], axis=0)
    egoB = jnp.concatenate([user_emb_b, item_emb_b], axis=0)
    colsA, rowsA, valsA = _pad_edges(adj_a_idx, adj_a_val)
    colsB, rowsB, valsB = _pad_edges(adj_b_idx, adj_b_val)
    zeros = jnp.zeros((N, HALF), jnp.float32)
    sumA, sumB = egoA, egoB
    for _ in range(LAYERS):
        egoA = _combine(_spmm(colsA, rowsA, valsA, egoA, zeros), egoA)
        egoB = _combine(_spmm(colsB, rowsB, valsB, egoB, zeros), egoB)
        egoA, egoB = _transfer(egoA, egoB)
        sumA = sumA + egoA
        sumB = sumB + egoB
    embsA = sumA / (LAYERS + 1)
    embsB = sumB / (LAYERS + 1)
    ua, ia = embsA[:N_USER], embsA[N_USER:]
    ub, ib = embsB[:N_USER], embsB[N_USER:]
    pos_a = jnp.sum(ua[data_a[0]] * ia[data_a[1]], axis=-1)
    neg_a = jnp.sum(ua[data_a[0]] * ia[data_a[2]], axis=-1)
    loss = jnp.mean(jax.nn.softplus(neg_a - pos_a))
    pos_b = jnp.sum(ub[data_b[0]] * ib[data_b[1]], axis=-1)
    neg_b = jnp.sum(ub[data_b[0]] * ib[data_b[2]], axis=-1)
    loss = loss + jnp.mean(jax.nn.softplus(neg_b - pos_b))
    return loss


# speedup vs baseline: 7.5871x; 2.9071x over previous
import functools

import jax
import jax.numpy as jnp
from jax import lax
from jax.experimental import pallas as pl
from jax.experimental.pallas import tpu as pltpu
from jax.experimental.pallas import tpu_sc as plsc

N_USER = 25000
N_OVERLAP = 10000
EMB = 64
LAYERS = 3
N = 50000
LAMBDA_A = 0.7
LAMBDA_B = 0.7

NNZ = 800000
HALF = 32
CHUNK = 128
NS = 16
NC = 2
SUP = 2
NSTEP = 200
NCT = SUP * NSTEP
NNZ_PAD = NS * NCT * CHUNK
RPT = 3000


def _make_spmm_sc():
    mesh = plsc.VectorSubcoreMesh(core_axis_name="c", subcore_axis_name="s",
                                  num_cores=NC, num_subcores=NS)

    @functools.partial(
        pl.kernel,
        out_type=jax.ShapeDtypeStruct((NC * N, HALF), jnp.float32),
        mesh=mesh,
        compiler_params=pltpu.CompilerParams(use_tc_tiling_on_sc=False),
        scratch_types=[
            pltpu.VMEM_SHARED((N, HALF), jnp.float32),
            pltpu.VMEM((2, SUP, CHUNK), jnp.int32),
            pltpu.VMEM((2, SUP, CHUNK), jnp.int32),
            pltpu.VMEM((2, SUP, CHUNK), jnp.float32),
            pltpu.VMEM((2, SUP, CHUNK, HALF), jnp.float32),
            pltpu.SemaphoreType.DMA,
            pltpu.SemaphoreType.DMA,
            pltpu.SemaphoreType.DMA,
            pltpu.SemaphoreType.DMA,
            pltpu.SemaphoreType.DMA,
            pltpu.SemaphoreType.DMA,
        ],
    )
    def spmm(ego_hbm, cols_hbm, rows_hbm, vals_hbm, zeros_hbm, out_hbm,
             acc, cb, rb, vb, db, isem0, isem1, gsem0, gsem1, ssem0, ssem1):
        kc = lax.axis_index("c")
        t = lax.axis_index("s")
        r0 = t * RPT
        pltpu.sync_copy(zeros_hbm.at[pl.ds(r0, RPT)], acc.at[pl.ds(r0, RPT)])

        @pl.when(t < 2)
        def _():
            rx = NS * RPT + t * 1000
            pltpu.sync_copy(zeros_hbm.at[pl.ds(rx, 1000)],
                            acc.at[pl.ds(rx, 1000)])
        plsc.subcore_barrier()
        koff = kc * N
        isems = (isem0, isem1)
        gsems = (gsem0, gsem1)
        ssems = (ssem0, ssem1)

        def idx_load(p, s):
            row = t * NCT + s * SUP
            sl = pl.ds(row, SUP)
            return [
                pltpu.async_copy(cols_hbm.at[sl], cb.at[p], isems[p]),
                pltpu.async_copy(rows_hbm.at[sl], rb.at[p], isems[p]),
                pltpu.async_copy(vals_hbm.at[sl], vb.at[p], isems[p]),
            ]

        def add_koff(p):
            def kbody(i, _):
                b = i // (CHUNK // 16)
                g = i % (CHUNK // 16)
                cb[p, b, pl.ds(g * 16, 16)] = cb[p, b, pl.ds(g * 16, 16)] + koff
                return 0
            lax.fori_loop(0, SUP * (CHUNK // 16), kbody, 0)

        def gathers(p):
            for b in range(SUP):
                pltpu.async_copy(ego_hbm.at[cb.at[p, b]], db.at[p, b],
                                 gsems[p])

        def drain(sem, n):
            for _ in range(n):
                pltpu.make_async_copy(zeros_hbm.at[pl.ds(0, CHUNK)],
                                      db.at[0, 0], sem).wait()

        def scale(p):
            for b in range(SUP):
                def grp(g, _):
                    v = vb[p, b, pl.ds(g * 16, 16)]
                    for j in range(16):
                        e = g * 16 + j
                        s = v[j]
                        db[p, b, e, pl.ds(0, 16)] = db[p, b, e, pl.ds(0, 16)] * s
                        db[p, b, e, pl.ds(16, 16)] = db[p, b, e, pl.ds(16, 16)] * s
                    return 0
                lax.fori_loop(0, CHUNK // 16, grp, 0)

        def scatters(p):
            for b in range(SUP):
                pltpu.async_copy(db.at[p, b], acc.at[rb.at[p, b]],
                                 ssems[p], add=True)

        for d in idx_load(0, 0):
            d.wait()
        add_koff(0)
        gathers(0)
        for d in idx_load(1, 1):
            d.wait()
        add_koff(1)

        def body2(s, _):
            @pl.when(s > 0)
            def _():
                drain(ssems[1], SUP)
            gathers(1)
            drain(gsems[0], SUP)
            di = idx_load(0, s + 2)
            scale(0)
            scatters(0)
            for d in di:
                d.wait()
            add_koff(0)
            drain(ssems[0], SUP)
            gathers(0)
            drain(gsems[1], SUP)
            di = idx_load(1, s + 3)
            scale(1)
            scatters(1)
            for d in di:
                d.wait()
            add_koff(1)
            return 0
        lax.fori_loop(0, (NSTEP - 2) // 2, lambda i, c: body2(i * 2, c), 0)

        drain(ssems[1], SUP)
        gathers(1)
        drain(gsems[0], SUP)
        scale(0)
        scatters(0)
        drain(gsems[1], SUP)
        scale(1)
        scatters(1)
        drain(ssems[0], SUP)
        drain(ssems[1], SUP)
        plsc.subcore_barrier()
        pltpu.sync_copy(acc.at[pl.ds(r0, RPT)],
                        out_hbm.at[pl.ds(koff + r0, RPT)])

        @pl.when(t < 2)
        def _():
            rx = NS * RPT + t * 1000
            pltpu.sync_copy(acc.at[pl.ds(rx, 1000)],
                            out_hbm.at[pl.ds(koff + rx, 1000)])

    return spmm


_spmm_sc = _make_spmm_sc()


def _combine_body(side_ref, ego_ref, out_ref):
    s = side_ref[...]
    e = ego_ref[...]
    out_ref[...] = s + e * s


def _combine(side, ego):
    blk = 1000
    return pl.pallas_call(
        _combine_body,
        out_shape=jax.ShapeDtypeStruct((N, EMB), jnp.float32),
        grid=(N // blk,),
        in_specs=[
            pl.BlockSpec((blk, EMB), lambda i: (i, 0)),
            pl.BlockSpec((blk, EMB), lambda i: (i, 0)),
        ],
        out_specs=pl.BlockSpec((blk, EMB), lambda i: (i, 0)),
    )(side, ego)


def _spmm(cols, rows, vals, ego, zeros):
    ego_flat = jnp.concatenate([ego[:, :HALF], ego[:, HALF:]], axis=0)
    out = _spmm_sc(ego_flat, cols, rows, vals, zeros)
    return jnp.concatenate([out[:N], out[N:]], axis=1)


def _transfer(egoA, egoB):
    ua, ia = egoA[:N_USER], egoA[N_USER:]
    ub, ib = egoB[:N_USER], egoB[N_USER:]
    oua, dua = ua[:N_OVERLAP], ua[N_OVERLAP:]
    oub, dub = ub[:N_OVERLAP], ub[N_OVERLAP:]
    u_lap = 0.5 * oua + 0.5 * oub
    ua_lam = LAMBDA_A * oua + (1.0 - LAMBDA_A) * oub
    ub_lam = LAMBDA_B * oub + (1.0 - LAMBDA_B) * oua
    new_oua = (u_lap + ua_lam) / 2.0
    new_oub = (u_lap + ub_lam) / 2.0
    egoA = jnp.concatenate([new_oua, dua, ia], axis=0)
    egoB = jnp.concatenate([new_oub, dub, ib], axis=0)
    return egoA, egoB


def _pad_edges(idx, val):
    pad = NNZ_PAD - NNZ
    spread = (jnp.arange(pad, dtype=jnp.int32) * 64) % N
    cols = jnp.concatenate([idx[1].astype(jnp.int32), spread])
    rows = jnp.concatenate([idx[0].astype(jnp.int32), spread])
    vals = jnp.concatenate([val, jnp.zeros((pad,), jnp.float32)])
    return (cols.reshape(-1, CHUNK), rows.reshape(-1, CHUNK),
            vals.reshape(-1, CHUNK))


def kernel(user_emb_a, item_emb_a, user_emb_b, item_emb_b,
           adj_a_val, adj_b_val, adj_a_idx, adj_b_idx, data_a, data_b):
    egoA = jnp.concatenate([user_emb_a, item_emb_a], axis=0)
    egoB = jnp.concatenate([user_emb_b, item_emb_b], axis=0)
    colsA, rowsA, valsA = _pad_edges(adj_a_idx, adj_a_val)
    colsB, rowsB, valsB = _pad_edges(adj_b_idx, adj_b_val)
    zeros = jnp.zeros((N, HALF), jnp.float32)
    sumA, sumB = egoA, egoB
    for _ in range(LAYERS):
        egoA = _combine(_spmm(colsA, rowsA, valsA, egoA, zeros), egoA)
        egoB = _combine(_spmm(colsB, rowsB, valsB, egoB, zeros), egoB)
        egoA, egoB = _transfer(egoA, egoB)
        sumA = sumA + egoA
        sumB = sumB + egoB
    embsA = sumA / (LAYERS + 1)
    embsB = sumB / (LAYERS + 1)
    ua, ia = embsA[:N_USER], embsA[N_USER:]
    ub, ib = embsB[:N_USER], embsB[N_USER:]
    pos_a = jnp.sum(ua[data_a[0]] * ia[data_a[1]], axis=-1)
    neg_a = jnp.sum(ua[data_a[0]] * ia[data_a[2]], axis=-1)
    loss = jnp.mean(jax.nn.softplus(neg_a - pos_a))
    pos_b = jnp.sum(ub[data_b[0]] * ib[data_b[1]], axis=-1)
    neg_b = jnp.sum(ub[data_b[0]] * ib[data_b[2]], axis=-1)
    loss = loss + jnp.mean(jax.nn.softplus(neg_b - pos_b))
    return loss


# speedup vs baseline: 8.7002x; 1.1467x over previous
import functools

import jax
import jax.numpy as jnp
from jax import lax
from jax.experimental import pallas as pl
from jax.experimental.pallas import tpu as pltpu
from jax.experimental.pallas import tpu_sc as plsc

N_USER = 25000
N_OVERLAP = 10000
EMB = 64
LAYERS = 3
N = 50000
BATCH = 4096

NNZ = 800000
HALF = 32
CHUNK = 128
NS = 16
NC = 2
SUP = 2
NSTEP = 200
NCT = SUP * NSTEP
NNZ_PAD = NS * NCT * CHUNK
NROWS2D = NNZ_PAD // CHUNK
RPT = 3000
CCH = 200
SPT = BATCH // NS

_MESH = plsc.VectorSubcoreMesh(core_axis_name="c", subcore_axis_name="s",
                               num_cores=NC, num_subcores=NS)
_SC_PARAMS = pltpu.CompilerParams(use_tc_tiling_on_sc=False)


def _zero_acc(zeros_hbm, acc, t):
    pltpu.sync_copy(zeros_hbm, acc.at[pl.ds(t * RPT, RPT)])

    @pl.when(t < 2)
    def _():
        rx = NS * RPT + t * 1000
        pltpu.sync_copy(zeros_hbm.at[pl.ds(0, 1000)], acc.at[pl.ds(rx, 1000)])


def _spmm_phase(kc, t, ego_hbm, cols2, rows2, vals2, zeros_hbm,
                acc, cb, rb, vb, db, isems, gsems, ssems):
    def idx_load(p, s):
        row = t * NCT + s * SUP
        sl = pl.ds(row, SUP)
        return [
            pltpu.async_copy(cols2.at[kc, sl], cb.at[p], isems[p]),
            pltpu.async_copy(rows2.at[sl], rb.at[p], isems[p]),
            pltpu.async_copy(vals2.at[sl], vb.at[p], isems[p]),
        ]

    def gathers(p):
        for b in range(SUP):
            pltpu.async_copy(ego_hbm.at[cb.at[p, b]],
                             db.at[p, pl.ds(b * CHUNK, CHUNK)], gsems[p])

    def drain(sem, n):
        for _ in range(n):
            pltpu.make_async_copy(zeros_hbm.at[pl.ds(0, CHUNK)],
                                  db.at[0, pl.ds(0, CHUNK)], sem).wait()

    def scale(p):
        def grp(g, _):
            v = vb[p, g // (CHUNK // 16), pl.ds((g % (CHUNK // 16)) * 16, 16)]
            for j in range(16):
                e = g * 16 + j
                s = v[j]
                db[p, e, pl.ds(0, 16)] = db[p, e, pl.ds(0, 16)] * s
                db[p, e, pl.ds(16, 16)] = db[p, e, pl.ds(16, 16)] * s
            return 0
        lax.fori_loop(0, SUP * CHUNK // 16, grp, 0)

    def scatters(p):
        for b in range(SUP):
            pltpu.async_copy(db.at[p, pl.ds(b * CHUNK, CHUNK)],
                             acc.at[rb.at[p, b]], ssems[p], add=True)

    for d in idx_load(0, 0):
        d.wait()
    gathers(0)
    for d in idx_load(1, 1):
        d.wait()

    def body2(s, _):
        @pl.when(s > 0)
        def _():
            drain(ssems[1], SUP)
        gathers(1)
        drain(gsems[0], SUP)
        di = idx_load(0, s + 2)
        scale(0)
        scatters(0)
        for d in di:
            d.wait()
        drain(ssems[0], SUP)
        gathers(0)
        drain(gsems[1], SUP)
        di = idx_load(1, s + 3)
        scale(1)
        scatters(1)
        for d in di:
            d.wait()
        return 0
    lax.fori_loop(0, (NSTEP - 2) // 2, lambda i, c: body2(i * 2, c), 0)

    drain(ssems[1], SUP)
    gathers(1)
    drain(gsems[0], SUP)
    scale(0)
    scatters(0)
    drain(gsems[1], SUP)
    scale(1)
    scatters(1)
    drain(ssems[0], SUP)
    drain(ssems[1], SUP)


def _ew_loop(dst, a, b, nrows, f):
    def body(i, _):
        r = i // (HALF // 16)
        h = (i % (HALF // 16)) * 16
        dst[r, pl.ds(h, 16)] = f(a[r, pl.ds(h, 16)], b[r, pl.ds(h, 16)])
        return 0
    lax.fori_loop(0, nrows * HALF // 16, body, 0)


def _combine_chunk(r, kc, layer, acc, src, sum_in, gcf_out, sum_out,
                   db0, db1, db2):
    ar = kc * N + r
    pltpu.sync_copy(acc.at[pl.ds(r, CCH)], db0)
    pltpu.sync_copy(src.at[pl.ds(ar, CCH)], db1)
    _ew_loop(db0, db0, db1, CCH, lambda s, e: s + e * s)
    pltpu.sync_copy(db0, gcf_out.at[pl.ds(ar, CCH)])

    @pl.when(r >= N_OVERLAP)
    def _():
        if layer == 0:
            _ew_loop(db1, db1, db0, CCH, lambda x, y: x + y)
            pltpu.sync_copy(db1, sum_out.at[pl.ds(ar, CCH)])
        else:
            pltpu.sync_copy(sum_in.at[pl.ds(ar, CCH)], db2)
            _ew_loop(db2, db2, db0, CCH, lambda x, y: x + y)
            pltpu.sync_copy(db2, sum_out.at[pl.ds(ar, CCH)])


def _transfer_chunk(r, kc, layer, srcA, srcB, sumA_in, sumB_in,
                    gcfA, gcfB, sumA_out, sumB_out, db0, db1, db2):
    ar = kc * N + r
    pltpu.sync_copy(gcfA.at[pl.ds(ar, CCH)], db0)
    pltpu.sync_copy(gcfB.at[pl.ds(ar, CCH)], db1)

    def mix(i, _):
        rr = i // (HALF // 16)
        h = (i % (HALF // 16)) * 16
        a = db0[rr, pl.ds(h, 16)]
        b = db1[rr, pl.ds(h, 16)]
        db0[rr, pl.ds(h, 16)] = 0.6 * a + 0.4 * b
        db1[rr, pl.ds(h, 16)] = 0.6 * b + 0.4 * a
        return 0
    lax.fori_loop(0, CCH * HALF // 16, mix, 0)
    pltpu.sync_copy(db0, gcfA.at[pl.ds(ar, CCH)])
    pltpu.sync_copy(db1, gcfB.at[pl.ds(ar, CCH)])
    for sin, sout, dnew in ((srcA if layer == 0 else sumA_in, sumA_out, db0),
                            (srcB if layer == 0 else sumB_in, sumB_out, db1)):
        pltpu.sync_copy(sin.at[pl.ds(ar, CCH)], db2)
        _ew_loop(db2, db2, dnew, CCH, lambda x, y: x + y)
        pltpu.sync_copy(db2, sout.at[pl.ds(ar, CCH)])


def _make_layer_kernel(layer):
    eg = jax.ShapeDtypeStruct((NC * N, HALF), jnp.float32)

    @functools.partial(
        pl.kernel,
        out_type=(eg, eg, eg, eg),
        mesh=_MESH,
        compiler_params=_SC_PARAMS,
        scratch_types=[
            pltpu.VMEM_SHARED((N, HALF), jnp.float32),
            pltpu.VMEM((2, SUP, CHUNK), jnp.int32),
            pltpu.VMEM((2, SUP, CHUNK), jnp.int32),
            pltpu.VMEM((2, SUP, CHUNK), jnp.float32),
            pltpu.VMEM((2, SUP * CHUNK, HALF), jnp.float32),
            pltpu.VMEM((CCH, HALF), jnp.float32),
            pltpu.SemaphoreType.DMA,
            pltpu.SemaphoreType.DMA,
            pltpu.SemaphoreType.DMA,
            pltpu.SemaphoreType.DMA,
            pltpu.SemaphoreType.DMA,
            pltpu.SemaphoreType.DMA,
        ],
    )
    def layer_kernel(egoA, egoB, colsA, rowsA, valsA, colsB, rowsB, valsB,
                     sumA_in, sumB_in, zeros_hbm,
                     gcfA, gcfB, sumA_out, sumB_out,
                     acc, cb, rb, vb, db, db2,
                     isem0, isem1, gsem0, gsem1, ssem0, ssem1):
        kc = lax.axis_index("c")
        t = lax.axis_index("s")
        isems = (isem0, isem1)
        gsems = (gsem0, gsem1)
        ssems = (ssem0, ssem1)
        db0 = db.at[0, pl.ds(0, CCH)]
        db1 = db.at[1, pl.ds(0, CCH)]

        for (src, cols2, rows2, vals2, gcf, sin, sout) in (
                (egoA, colsA, rowsA, valsA, gcfA, sumA_in, sumA_out),
                (egoB, colsB, rowsB, valsB, gcfB, sumB_in, sumB_out)):
            _zero_acc(zeros_hbm, acc, t)
            plsc.subcore_barrier()
            _spmm_phase(kc, t, src, cols2, rows2, vals2, zeros_hbm,
                        acc, cb, rb, vb, db, isems, gsems, ssems)
            plsc.subcore_barrier()

            def comb(i, _, base):
                _combine_chunk(base + i * CCH, kc, layer, acc, src, sin,
                               gcf, sout, db0, db1, db2)
                return 0
            lax.fori_loop(0, RPT // CCH,
                          functools.partial(comb, base=t * RPT), 0)

            @pl.when(t < 2)
            def _():
                lax.fori_loop(
                    0, 1000 // CCH,
                    functools.partial(comb, base=NS * RPT + t * 1000), 0)
            plsc.subcore_barrier()

        def trans(i, _, base):
            _transfer_chunk(base + i * CCH, kc, layer, egoA, egoB,
                            sumA_in, sumB_in, gcfA, gcfB, sumA_out, sumB_out,
                            db0, db1, db2)
            return 0
        lax.fori_loop(0, 3, functools.partial(trans, base=t * 600), 0)

        @pl.when(t < 2)
        def _():
            lax.fori_loop(0, 1,
                          functools.partial(trans, base=NS * 600 + t * 200), 0)

    return layer_kernel


_layer_first = _make_layer_kernel(0)
_layer_rest = _make_layer_kernel(1)


def _make_gather_kernel():
    @functools.partial(
        pl.kernel,
        out_type=jax.ShapeDtypeStruct((2, 3, NC, BATCH, HALF), jnp.float32),
        mesh=_MESH,
        compiler_params=_SC_PARAMS,
        scratch_types=[
            pltpu.VMEM((2, CHUNK), jnp.int32),
            pltpu.VMEM((SPT, HALF), jnp.float32),
            pltpu.SemaphoreType.DMA,
        ],
    )
    def gather_kernel(sumA, sumB, datb, out, ib, gb, sem):
        kc = lax.axis_index("c")
        t = lax.axis_index("s")
        for d, src in ((0, sumA), (1, sumB)):
            for kind in range(3):
                pltpu.sync_copy(datb.at[d, kind, kc, pl.ds(t * 2, 2)], ib)
                for b in range(2):
                    pltpu.async_copy(src.at[ib.at[b]],
                                     gb.at[pl.ds(b * CHUNK, CHUNK)],
                                     sem).wait()
                pltpu.sync_copy(gb, out.at[d, kind, kc, pl.ds(t * SPT, SPT)])

    return gather_kernel


_gather_k = _make_gather_kernel()


def _loss_body(g_ref, out_ref):
    g = g_ref[...]
    u = g[:, 0]
    p = g[:, 1]
    n = g[:, 2]
    pos = jnp.sum(u * p, axis=(1, 3)) / 16.0
    neg = jnp.sum(u * n, axis=(1, 3)) / 16.0
    per = jnp.mean(jax.nn.softplus(neg - pos), axis=1)
    out_ref[0, 0] = per[0] + per[1]


def _pad_edges(idx, val):
    pad = NNZ_PAD - NNZ
    spread = (jnp.arange(pad, dtype=jnp.int32) * 64) % N
    cols = jnp.concatenate([idx[1].astype(jnp.int32), spread])
    cols2 = jnp.stack([cols, cols + N]).reshape(NC, NROWS2D, CHUNK)
    rows = jnp.concatenate([idx[0].astype(jnp.int32), spread])
    vals = jnp.concatenate([val, jnp.zeros((pad,), jnp.float32)])
    return (cols2, rows.reshape(NROWS2D, CHUNK), vals.reshape(NROWS2D, CHUNK))


def kernel(user_emb_a, item_emb_a, user_emb_b, item_emb_b,
           adj_a_val, adj_b_val, adj_a_idx, adj_b_idx, data_a, data_b):
    egoA = jnp.concatenate(
        [jnp.concatenate([user_emb_a[:, :HALF], item_emb_a[:, :HALF]]),
         jnp.concatenate([user_emb_a[:, HALF:], item_emb_a[:, HALF:]])])
    egoB = jnp.concatenate(
        [jnp.concatenate([user_emb_b[:, :HALF], item_emb_b[:, :HALF]]),
         jnp.concatenate([user_emb_b[:, HALF:], item_emb_b[:, HALF:]])])
    colsA, rowsA, valsA = _pad_edges(adj_a_idx, adj_a_val)
    colsB, rowsB, valsB = _pad_edges(adj_b_idx, adj_b_val)
    dat = jnp.stack([data_a.astype(jnp.int32), data_b.astype(jnp.int32)])
    kind_bias = jnp.array([0, N_USER, N_USER], jnp.int32)[None, :, None]
    core_bias = jnp.array([0, N], jnp.int32)[None, None, :, None]
    datb = (dat + kind_bias)[:, :, None, :] + core_bias
    datb = datb.reshape(2, 3, NC, BATCH // CHUNK, CHUNK)
    zeros = jnp.zeros((RPT, HALF), jnp.float32)

    sumA = sumB = jnp.zeros((NC * N, HALF), jnp.float32)
    for layer in range(LAYERS):
        fn = _layer_first if layer == 0 else _layer_rest
        egoA, egoB, sumA, sumB = fn(egoA, egoB, colsA, rowsA, valsA,
                                    colsB, rowsB, valsB, sumA, sumB, zeros)
    gbuf = _gather_k(sumA, sumB, datb)
    loss = pl.pallas_call(
        _loss_body,
        out_shape=jax.ShapeDtypeStruct((1, 1), jnp.float32),
        out_specs=pl.BlockSpec(memory_space=pltpu.SMEM),
    )(gbuf)
    return loss[0, 0]
